# Initial kernel scaffold; baseline (speedup 1.0000x reference)
import jax
import jax.numpy as jnp
from jax.experimental import pallas as pl


def kernel(x, edge_index, lin_src, att_src, att_dst, bias):
    raise NotImplementedError("write your pallas kernel here")


# speedup vs baseline: 21.6170x; 21.6170x over previous
import functools

import jax
import jax.numpy as jnp
from jax import lax
from jax.experimental import pallas as pl
from jax.experimental.pallas import tpu as pltpu
from jax.experimental.pallas import tpu_sc as plsc

NC = 2
NS = 16
NW = NC * NS
L = 16


def _k1_proj(x_pad, lin, ats, atd, Np, D, C, R):
    G = Np // R

    def body(x_ref, lin_ref, ats_ref, atd_ref,
             xp_ref, as_ref, ad_ref, m_ref, asacc, adacc):
        i = pl.program_id(0)
        xp = jnp.dot(x_ref[...], lin_ref[...],
                     preferred_element_type=jnp.float32)
        xp_ref[...] = xp
        a_s = jnp.sum(xp * ats_ref[...], axis=1).reshape(R // 128, 128)
        a_d = jnp.sum(xp * atd_ref[...], axis=1).reshape(R // 128, 128)
        as_ref[...] = a_s
        ad_ref[...] = a_d

        @pl.when(i == 0)
        def _():
            asacc[...] = jnp.full((8, 128), -jnp.inf, jnp.float32)
            adacc[...] = jnp.full((8, 128), -jnp.inf, jnp.float32)

        asacc[...] = jnp.maximum(asacc[...], a_s)
        adacc[...] = jnp.maximum(adacc[...], a_d)

        @pl.when(i == G - 1)
        def _():
            m = jnp.maximum(jnp.max(asacc[...]) + jnp.max(adacc[...]), 0.0)
            m_ref[...] = jnp.full((8, 128), m, jnp.float32)

    return pl.pallas_call(
        body,
        grid=(G,),
        in_specs=[
            pl.BlockSpec((R, D), lambda i: (i, 0)),
            pl.BlockSpec((D, C), lambda i: (0, 0)),
            pl.BlockSpec((1, C), lambda i: (0, 0)),
            pl.BlockSpec((1, C), lambda i: (0, 0)),
        ],
        out_specs=[
            pl.BlockSpec((R, C), lambda i: (i, 0)),
            pl.BlockSpec((R // 128, 128), lambda i: (i, 0)),
            pl.BlockSpec((R // 128, 128), lambda i: (i, 0)),
            pl.BlockSpec((8, 128), lambda i: (0, 0)),
        ],
        out_shape=[
            jax.ShapeDtypeStruct((Np, C), jnp.float32),
            jax.ShapeDtypeStruct((Np // 128, 128), jnp.float32),
            jax.ShapeDtypeStruct((Np // 128, 128), jnp.float32),
            jax.ShapeDtypeStruct((8, 128), jnp.float32),
        ],
        scratch_shapes=[
            pltpu.VMEM((8, 128), jnp.float32),
            pltpu.VMEM((8, 128), jnp.float32),
        ],
    )(x_pad, lin, ats, atd)


def _k2_edge_ex(src3, dst3, as_flat, ad_flat, m_flat, Np, NB, B):
    mesh = plsc.VectorSubcoreMesh(core_axis_name="c", subcore_axis_name="s")
    rows_per_tile = Np // NS

    @functools.partial(
        pl.kernel,
        mesh=mesh,
        out_type=[
            jax.ShapeDtypeStruct((NW, NB, B), jnp.float32),
            jax.ShapeDtypeStruct((NC, Np), jnp.float32),
        ],
        scratch_types=[
            pltpu.VMEM((NB, B), jnp.int32),
            pltpu.VMEM((NB, B), jnp.int32),
            pltpu.VMEM((Np,), jnp.float32),
            pltpu.VMEM((Np,), jnp.float32),
            pltpu.VMEM((NB, B), jnp.float32),
            pltpu.VMEM((L,), jnp.float32),
            pltpu.VMEM((rows_per_tile,), jnp.float32),
            pltpu.VMEM_SHARED((Np,), jnp.float32),
        ],
        compiler_params=pltpu.CompilerParams(needs_layout_passes=False),
    )
    def k2(src_h, dst_h, as_h, ad_h, m_h, ex_h, den_h,
           src_v, dst_v, as_v, ad_v, ex_v, m_v, zero_v, den_sh):
        c = lax.axis_index("c")
        s = lax.axis_index("s")
        wid = c * NS + s
        pltpu.sync_copy(src_h.at[wid], src_v)
        pltpu.sync_copy(dst_h.at[wid], dst_v)
        pltpu.sync_copy(as_h, as_v)
        pltpu.sync_copy(ad_h, ad_v)
        pltpu.sync_copy(m_h.at[pl.ds(0, L)], m_v)

        def zero_body(j, _):
            zero_v[pl.ds(j * L, L)] = jnp.zeros((L,), jnp.float32)
            return _

        lax.fori_loop(0, rows_per_tile // L, zero_body, 0)
        pltpu.sync_copy(zero_v, den_sh.at[pl.ds(s * rows_per_tile,
                                                rows_per_tile)])
        plsc.subcore_barrier()

        mval = m_v[...]

        def blk(b, _):
            for g in range(B // L):
                sv = src_v[b, pl.ds(g * L, L)]
                dv = dst_v[b, pl.ds(g * L, L)]
                asg = plsc.load_gather(as_v, [sv])
                adg = plsc.load_gather(ad_v, [dv])
                al = asg + adg
                al = jnp.where(al >= 0.0, al, 0.2 * al) - mval
                exv = jnp.exp(al)
                exv = jnp.where(sv != dv, exv, 0.0)
                ex_v[b, pl.ds(g * L, L)] = exv
            pltpu.sync_copy(ex_v.at[b], den_sh.at[dst_v.at[b]], add=True)
            return _

        lax.fori_loop(0, NB, blk, 0)
        pltpu.sync_copy(ex_v, ex_h.at[wid])
        plsc.subcore_barrier()
        pltpu.sync_copy(
            den_sh.at[pl.ds(s * rows_per_tile, rows_per_tile)],
            den_h.at[c, pl.ds(s * rows_per_tile, rows_per_tile)])

    return k2(src3, dst3, as_flat, ad_flat, m_flat)


def _k3_denom(den2, as2, ad2, m2, Np):
    G = Np // 128 // 8

    def body(d2_ref, as_ref, ad_ref, m_ref, dt_ref, sa_ref):
        a = as_ref[...] + ad_ref[...]
        a = jnp.where(a >= 0.0, a, 0.2 * a) - m_ref[...]
        exs = jnp.exp(a)
        d2 = d2_ref[...]
        dt = d2[0] + d2[1] + exs
        dt_ref[...] = dt
        sa_ref[...] = exs / (dt + 1e-16)

    return pl.pallas_call(
        body,
        grid=(G,),
        in_specs=[
            pl.BlockSpec((2, 8, 128), lambda i: (0, i, 0)),
            pl.BlockSpec((8, 128), lambda i: (i, 0)),
            pl.BlockSpec((8, 128), lambda i: (i, 0)),
            pl.BlockSpec((8, 128), lambda i: (0, 0)),
        ],
        out_specs=[
            pl.BlockSpec((8, 128), lambda i: (i, 0)),
            pl.BlockSpec((8, 128), lambda i: (i, 0)),
        ],
        out_shape=[
            jax.ShapeDtypeStruct((Np // 128, 128), jnp.float32),
            jax.ShapeDtypeStruct((Np // 128, 128), jnp.float32),
        ],
    )(den2, as2, ad2, m2)


def _k4_aggregate(src3, dst3, ex3, dt_flat, xp, Np, NB, B, C, SB):
    mesh = plsc.VectorSubcoreMesh(core_axis_name="c", subcore_axis_name="s")
    rows_per_tile = Np // NS
    NST = NB // SB

    @functools.partial(
        pl.kernel,
        mesh=mesh,
        out_type=[
            jax.ShapeDtypeStruct((NC, Np, C), jnp.float32),
        ],
        scratch_types=[
            pltpu.VMEM((SB, B), jnp.int32),
            pltpu.VMEM((SB, B), jnp.int32),
            pltpu.VMEM((SB, B), jnp.float32),
            pltpu.VMEM((Np,), jnp.float32),
            pltpu.VMEM((B, C), jnp.float32),
            pltpu.VMEM((B,), jnp.float32),
            pltpu.VMEM_SHARED((Np, C), jnp.float32),
            pltpu.SemaphoreType.DMA,
        ],
        compiler_params=pltpu.CompilerParams(needs_layout_passes=False),
    )
    def k4(src_h, dst_h, ex_h, dt_h, xp_h, out_h,
           src_v, dst_v, ex_v, dt_v, rows_v, alpha_v, out_sh, sem):
        c = lax.axis_index("c")
        s = lax.axis_index("s")
        wid = c * NS + s
        pltpu.sync_copy(dt_h, dt_v)

        def zrow(r, _):
            for j in range(C // L):
                rows_v[r, pl.ds(j * L, L)] = jnp.zeros((L,), jnp.float32)
            return _

        lax.fori_loop(0, B, zrow, 0)
        for k in range(rows_per_tile // B):
            pltpu.sync_copy(
                rows_v, out_sh.at[pl.ds(s * rows_per_tile + k * B, B)])
        plsc.subcore_barrier()

        def stage(t, _):
            pltpu.sync_copy(src_h.at[wid, t], src_v)
            pltpu.sync_copy(dst_h.at[wid, t], dst_v)
            pltpu.sync_copy(ex_h.at[wid, t], ex_v)

            def blk(b, _1):
                cp = pltpu.async_copy(xp_h.at[src_v.at[b]], rows_v, sem)
                for g in range(B // L):
                    dv = dst_v[b, pl.ds(g * L, L)]
                    dtg = plsc.load_gather(dt_v, [dv])
                    exg = ex_v[b, pl.ds(g * L, L)]
                    alpha_v[pl.ds(g * L, L)] = exg / (dtg + 1e-16)
                cp.wait()

                def scale(e, _2):
                    av = plsc.load_gather(
                        alpha_v, [jnp.full((L,), e, jnp.int32)])
                    for j in range(C // L):
                        rows_v[e, pl.ds(j * L, L)] = (
                            rows_v[e, pl.ds(j * L, L)] * av)
                    return _2

                lax.fori_loop(0, B, scale, 0)
                pltpu.sync_copy(rows_v, out_sh.at[dst_v.at[b]], add=True)
                return _1

            lax.fori_loop(0, SB, blk, 0)
            return _

        lax.fori_loop(0, NST, stage, 0)
        plsc.subcore_barrier()
        pltpu.sync_copy(
            out_sh.at[pl.ds(s * rows_per_tile, rows_per_tile)],
            out_h.at[c, pl.ds(s * rows_per_tile, rows_per_tile)])

    return k4(src3, dst3, ex3, dt_flat, xp)


def _k5_combine(outp, xp, sa_col, bias2, Np, C, R):
    G = Np // R

    def body(p_ref, xp_ref, sa_ref, b_ref, o_ref):
        p = p_ref[...]
        o_ref[...] = p[0] + p[1] + xp_ref[...] * sa_ref[...] + b_ref[...]

    return pl.pallas_call(
        body,
        grid=(G,),
        in_specs=[
            pl.BlockSpec((2, R, C), lambda i: (0, i, 0)),
            pl.BlockSpec((R, C), lambda i: (i, 0)),
            pl.BlockSpec((R, 1), lambda i: (i, 0)),
            pl.BlockSpec((1, C), lambda i: (0, 0)),
        ],
        out_specs=pl.BlockSpec((R, C), lambda i: (i, 0)),
        out_shape=jax.ShapeDtypeStruct((Np, C), jnp.float32),
    )(outp, xp, sa_col, bias2)


def kernel(x, edge_index, lin_src, att_src, att_dst, bias):
    N, D = x.shape
    C = lin_src.shape[1]
    H = att_src.shape[1]
    E = edge_index.shape[1]
    Np = ((N + 1023) // 1024) * 1024
    EW = E // NW
    B = 80
    NB = EW // B
    assert EW * NW == E and NB * B == EW and C % L == 0 and H == 1

    x_pad = jnp.pad(x, ((0, Np - N), (0, 0)))
    src3 = edge_index[0].astype(jnp.int32).reshape(NW, NB, B)
    dst3 = edge_index[1].astype(jnp.int32).reshape(NW, NB, B)
    ats = att_src.reshape(1, C)
    atd = att_dst.reshape(1, C)

    xp, as2, ad2, m2 = _k1_proj(x_pad, lin_src, ats, atd, Np, D, C, 1024)
    ex3, den2 = _k2_edge_ex(src3, dst3, as2.reshape(Np), ad2.reshape(Np),
                            m2.reshape(8 * 128), Np, NB, B)
    dt2, sa2 = _k3_denom(den2.reshape(NC, Np // 128, 128), as2, ad2, m2, Np)
    SB = 25
    src4 = src3.reshape(NW, NB // SB, SB, B)
    dst4 = dst3.reshape(NW, NB // SB, SB, B)
    ex4 = ex3.reshape(NW, NB // SB, SB, B)
    (outp,) = _k4_aggregate(src4, dst4, ex4, dt2.reshape(Np), xp,
                            Np, NB, B, C, SB)
    out = _k5_combine(outp, xp, sa2.reshape(Np, 1), bias.reshape(1, C),
                      Np, C, 256)
    return out[:N]


# speedup vs baseline: 28.6459x; 1.3252x over previous
import functools

import jax
import jax.numpy as jnp
from jax import lax
from jax.experimental import pallas as pl
from jax.experimental.pallas import tpu as pltpu
from jax.experimental.pallas import tpu_sc as plsc

NC = 2
NS = 16
NW = NC * NS
L = 16


def _k1_proj(x_pad, lin, ats, atd, Np, D, C, R):
    G = Np // R

    def body(x_ref, lin_ref, ats_ref, atd_ref,
             xp_ref, as_ref, ad_ref, m_ref, asacc, adacc):
        i = pl.program_id(0)
        xp = jnp.dot(x_ref[...], lin_ref[...],
                     preferred_element_type=jnp.float32)
        xp_ref[...] = xp
        a_s = jnp.sum(xp * ats_ref[...], axis=1).reshape(R // 128, 128)
        a_d = jnp.sum(xp * atd_ref[...], axis=1).reshape(R // 128, 128)
        as_ref[...] = a_s
        ad_ref[...] = a_d

        @pl.when(i == 0)
        def _():
            asacc[...] = jnp.full((8, 128), -jnp.inf, jnp.float32)
            adacc[...] = jnp.full((8, 128), -jnp.inf, jnp.float32)

        asacc[...] = jnp.maximum(asacc[...], a_s)
        adacc[...] = jnp.maximum(adacc[...], a_d)

        @pl.when(i == G - 1)
        def _():
            m = jnp.maximum(jnp.max(asacc[...]) + jnp.max(adacc[...]), 0.0)
            m_ref[...] = jnp.full((8, 128), m, jnp.float32)

    return pl.pallas_call(
        body,
        grid=(G,),
        in_specs=[
            pl.BlockSpec((R, D), lambda i: (i, 0)),
            pl.BlockSpec((D, C), lambda i: (0, 0)),
            pl.BlockSpec((1, C), lambda i: (0, 0)),
            pl.BlockSpec((1, C), lambda i: (0, 0)),
        ],
        out_specs=[
            pl.BlockSpec((R, C), lambda i: (i, 0)),
            pl.BlockSpec((R // 128, 128), lambda i: (i, 0)),
            pl.BlockSpec((R // 128, 128), lambda i: (i, 0)),
            pl.BlockSpec((8, 128), lambda i: (0, 0)),
        ],
        out_shape=[
            jax.ShapeDtypeStruct((Np, C), jnp.float32),
            jax.ShapeDtypeStruct((Np // 128, 128), jnp.float32),
            jax.ShapeDtypeStruct((Np // 128, 128), jnp.float32),
            jax.ShapeDtypeStruct((8, 128), jnp.float32),
        ],
        scratch_shapes=[
            pltpu.VMEM((8, 128), jnp.float32),
            pltpu.VMEM((8, 128), jnp.float32),
        ],
    )(x_pad, lin, ats, atd)


def _k2_edge_ex(src3, dst3, as_flat, ad_flat, m_flat, Np, NB, B):
    mesh = plsc.VectorSubcoreMesh(core_axis_name="c", subcore_axis_name="s")
    rows_per_tile = Np // NS

    @functools.partial(
        pl.kernel,
        mesh=mesh,
        out_type=[
            jax.ShapeDtypeStruct((NW, NB, B), jnp.float32),
            jax.ShapeDtypeStruct((NC, Np), jnp.float32),
        ],
        scratch_types=[
            pltpu.VMEM((NB, B), jnp.int32),
            pltpu.VMEM((NB, B), jnp.int32),
            pltpu.VMEM((Np,), jnp.float32),
            pltpu.VMEM((Np,), jnp.float32),
            pltpu.VMEM((NB, B), jnp.float32),
            pltpu.VMEM((L,), jnp.float32),
            pltpu.VMEM((rows_per_tile,), jnp.float32),
            pltpu.VMEM_SHARED((Np,), jnp.float32),
        ],
        compiler_params=pltpu.CompilerParams(needs_layout_passes=False),
    )
    def k2(src_h, dst_h, as_h, ad_h, m_h, ex_h, den_h,
           src_v, dst_v, as_v, ad_v, ex_v, m_v, zero_v, den_sh):
        c = lax.axis_index("c")
        s = lax.axis_index("s")
        wid = c * NS + s
        pltpu.sync_copy(src_h.at[wid], src_v)
        pltpu.sync_copy(dst_h.at[wid], dst_v)
        pltpu.sync_copy(as_h, as_v)
        pltpu.sync_copy(ad_h, ad_v)
        pltpu.sync_copy(m_h.at[pl.ds(0, L)], m_v)

        def zero_body(j, _):
            zero_v[pl.ds(j * L, L)] = jnp.zeros((L,), jnp.float32)
            return _

        lax.fori_loop(0, rows_per_tile // L, zero_body, 0)
        pltpu.sync_copy(zero_v, den_sh.at[pl.ds(s * rows_per_tile,
                                                rows_per_tile)])
        plsc.subcore_barrier()

        mval = m_v[...]

        def blk(b, _):
            for g in range(B // L):
                sv = src_v[b, pl.ds(g * L, L)]
                dv = dst_v[b, pl.ds(g * L, L)]
                asg = plsc.load_gather(as_v, [sv])
                adg = plsc.load_gather(ad_v, [dv])
                al = asg + adg
                al = jnp.where(al >= 0.0, al, 0.2 * al) - mval
                exv = jnp.exp(al)
                exv = jnp.where(sv != dv, exv, 0.0)
                ex_v[b, pl.ds(g * L, L)] = exv
            pltpu.sync_copy(ex_v.at[b], den_sh.at[dst_v.at[b]], add=True)
            return _

        lax.fori_loop(0, NB, blk, 0)
        pltpu.sync_copy(ex_v, ex_h.at[wid])
        plsc.subcore_barrier()
        pltpu.sync_copy(
            den_sh.at[pl.ds(s * rows_per_tile, rows_per_tile)],
            den_h.at[c, pl.ds(s * rows_per_tile, rows_per_tile)])

    return k2(src3, dst3, as_flat, ad_flat, m_flat)


def _k4_aggregate(src4, dst4, ex4, xp, Np, NB, B, C, SB):
    mesh = plsc.VectorSubcoreMesh(core_axis_name="c", subcore_axis_name="s")
    rows_per_tile = Np // NS
    NST = NB // SB
    PAIRS = (SB - 1) // 2
    assert SB % 2 == 1

    @functools.partial(
        pl.kernel,
        mesh=mesh,
        out_type=[
            jax.ShapeDtypeStruct((NC, Np, C), jnp.float32),
        ],
        scratch_types=[
            pltpu.VMEM((SB, B), jnp.int32),
            pltpu.VMEM((SB, B), jnp.int32),
            pltpu.VMEM((SB, B), jnp.float32),
            pltpu.VMEM((B, C), jnp.float32),
            pltpu.VMEM((B, C), jnp.float32),
            pltpu.VMEM_SHARED((Np, C), jnp.float32),
            pltpu.SemaphoreType.DMA,
            pltpu.SemaphoreType.DMA,
            pltpu.SemaphoreType.DMA,
            pltpu.SemaphoreType.DMA,
        ],
        compiler_params=pltpu.CompilerParams(needs_layout_passes=False),
    )
    def k4(src_h, dst_h, ex_h, xp_h, out_h,
           src_v, dst_v, ex_v, rows_a, rows_b,
           out_sh, sem_ga, sem_gb, sem_sa, sem_sb):
        c = lax.axis_index("c")
        s = lax.axis_index("s")
        wid = c * NS + s

        def zrow(r, _):
            for j in range(C // L):
                rows_a[r, pl.ds(j * L, L)] = jnp.zeros((L,), jnp.float32)
            return _

        lax.fori_loop(0, B, zrow, 0)
        for k in range(rows_per_tile // B):
            pltpu.sync_copy(
                rows_a, out_sh.at[pl.ds(s * rows_per_tile + k * B, B)])
        plsc.subcore_barrier()

        def g_start(b, rows, sem):
            pltpu.make_async_copy(xp_h.at[src_v.at[b]], rows, sem).start()

        def g_wait(b, rows, sem):
            pltpu.make_async_copy(xp_h.at[src_v.at[b]], rows, sem).wait()

        def s_start(b, rows, sem):
            pltpu.make_async_copy(
                rows, out_sh.at[dst_v.at[b]], sem).start(add=True)

        def s_wait(b, rows, sem):
            pltpu.make_async_copy(rows, out_sh.at[dst_v.at[b]], sem).wait()

        def scale(rows, b):
            bidx = jnp.full((L,), b, jnp.int32)

            def sbody(j, _2):
                for u in range(2):
                    e = 2 * j + u
                    av = plsc.load_gather(
                        ex_v, [bidx, jnp.full((L,), e, jnp.int32)])
                    for q in range(C // L):
                        rows[e, pl.ds(q * L, L)] = (
                            rows[e, pl.ds(q * L, L)] * av)
                return _2

            lax.fori_loop(0, B // 2, sbody, 0)

        def stage(t, _):
            pltpu.sync_copy(src_h.at[wid, t], src_v)
            pltpu.sync_copy(dst_h.at[wid, t], dst_v)
            pltpu.sync_copy(ex_h.at[wid, t], ex_v)
            g_start(0, rows_a, sem_ga)

            def pair(i, _1):
                b0 = 2 * i
                b1 = 2 * i + 1
                g_wait(b0, rows_a, sem_ga)
                g_start(b1, rows_b, sem_gb)
                scale(rows_a, b0)
                s_start(b0, rows_a, sem_sa)
                g_wait(b1, rows_b, sem_gb)
                scale(rows_b, b1)
                s_start(b1, rows_b, sem_sb)
                s_wait(b0, rows_a, sem_sa)
                g_start(b1 + 1, rows_a, sem_ga)
                s_wait(b1, rows_b, sem_sb)
                return _1

            lax.fori_loop(0, PAIRS, pair, 0)
            b_last = SB - 1
            g_wait(b_last, rows_a, sem_ga)
            scale(rows_a, b_last)
            s_start(b_last, rows_a, sem_sa)
            s_wait(b_last, rows_a, sem_sa)
            return _

        lax.fori_loop(0, NST, stage, 0)
        plsc.subcore_barrier()
        pltpu.sync_copy(
            out_sh.at[pl.ds(s * rows_per_tile, rows_per_tile)],
            out_h.at[c, pl.ds(s * rows_per_tile, rows_per_tile)])

    return k4(src4, dst4, ex4, xp)


def _k5_combine(outp, xp, as_col, ad_col, d0_col, d1_col, m_sc, bias2,
                Np, C, R):
    G = Np // R

    def body(p_ref, xp_ref, as_ref, ad_ref, d0_ref, d1_ref, m_ref, b_ref,
             o_ref):
        a = as_ref[...] + ad_ref[...]
        a = jnp.where(a >= 0.0, a, 0.2 * a) - m_ref[...]
        exs = jnp.exp(a)
        dt = d0_ref[...] + d1_ref[...] + exs
        inv = 1.0 / (dt + 1e-16)
        p = p_ref[...]
        o_ref[...] = (p[0] + p[1] + xp_ref[...] * exs) * inv + b_ref[...]

    return pl.pallas_call(
        body,
        grid=(G,),
        in_specs=[
            pl.BlockSpec((2, R, C), lambda i: (0, i, 0)),
            pl.BlockSpec((R, C), lambda i: (i, 0)),
            pl.BlockSpec((R, 1), lambda i: (i, 0)),
            pl.BlockSpec((R, 1), lambda i: (i, 0)),
            pl.BlockSpec((R, 1), lambda i: (i, 0)),
            pl.BlockSpec((R, 1), lambda i: (i, 0)),
            pl.BlockSpec((1, 1), lambda i: (0, 0)),
            pl.BlockSpec((1, C), lambda i: (0, 0)),
        ],
        out_specs=pl.BlockSpec((R, C), lambda i: (i, 0)),
        out_shape=jax.ShapeDtypeStruct((Np, C), jnp.float32),
    )(outp, xp, as_col, ad_col, d0_col, d1_col, m_sc, bias2)


def kernel(x, edge_index, lin_src, att_src, att_dst, bias):
    N, D = x.shape
    C = lin_src.shape[1]
    H = att_src.shape[1]
    E = edge_index.shape[1]
    Np = ((N + 1023) // 1024) * 1024
    EW = E // NW
    B = 80
    NB = EW // B
    assert EW * NW == E and NB * B == EW and C % L == 0 and H == 1

    x_pad = jnp.pad(x, ((0, Np - N), (0, 0)))
    src3 = edge_index[0].astype(jnp.int32).reshape(NW, NB, B)
    dst3 = edge_index[1].astype(jnp.int32).reshape(NW, NB, B)
    ats = att_src.reshape(1, C)
    atd = att_dst.reshape(1, C)

    xp, as2, ad2, m2 = _k1_proj(x_pad, lin_src, ats, atd, Np, D, C, 1024)
    m_flat = m2.reshape(8 * 128)
    ex3, den2 = _k2_edge_ex(src3, dst3, as2.reshape(Np), ad2.reshape(Np),
                            m_flat, Np, NB, B)
    SB = 25
    src4 = src3.reshape(NW, NB // SB, SB, B)
    dst4 = dst3.reshape(NW, NB // SB, SB, B)
    ex4 = ex3.reshape(NW, NB // SB, SB, B)
    (outp,) = _k4_aggregate(src4, dst4, ex4, xp, Np, NB, B, C, SB)
    out = _k5_combine(outp, xp, as2.reshape(Np, 1), ad2.reshape(Np, 1),
                      den2[0].reshape(Np, 1), den2[1].reshape(Np, 1),
                      m_flat[:1].reshape(1, 1), bias.reshape(1, C),
                      Np, C, 256)
    return out[:N]
